# baseline (speedup 1.0000x reference)
import functools
import math

import jax
import jax.numpy as jnp
from jax import lax
from jax.experimental import pallas as pl
from jax.experimental.pallas import tpu as pltpu
from jax.experimental.pallas import tpu_sc as plsc

_TN = 2048
_TR = 1024
_CH = 64


def _attn_body(ridx_ref, nm_ref, mpT_ref, sp_ref, wff_ref, wq_ref, wk_ref,
               wv_ref, g1_ref, b1_ref,
               v_out, wT_out, iT_out, f0_out, f1_out, w0_out, w1_out, k_scr):
    b = pl.program_id(0)
    nb = pl.program_id(1)
    K, C = k_scr.shape
    TN = w0_out.shape[2]

    @pl.when(nb == 0)
    def _():
        sp = sp_ref[0]
        k_scr[...] = lax.dot_general(sp, wk_ref[...], (((1,), (1,)), ((), ())),
                                     preferred_element_type=jnp.float32)
        v_out[0] = lax.dot_general(sp, wv_ref[...], (((1,), (1,)), ((), ())),
                                   preferred_element_type=jnp.float32)

    Lb = jnp.maximum(jnp.sum(nm_ref[pl.ds(b, 1), :]), 1.0)
    denom = jnp.maximum(Lb - 1.0, 1.0)
    m_row = nm_ref[pl.ds(b, 1), pl.ds(nb * TN, TN)]
    pos = jnp.clip(ridx_ref[pl.ds(b, 1), pl.ds(nb * TN, TN)] / denom, 0.0, 1.0)
    projT = (2.0 * math.pi) * (wff_ref[...] * pos)
    q0T = jnp.concatenate([jnp.cos(projT), jnp.sin(projT)], axis=0)
    q0T = q0T * m_row
    mu = jnp.mean(q0T, axis=0, keepdims=True)
    var = jnp.mean((q0T - mu) ** 2, axis=0, keepdims=True)
    q0T = (q0T - mu) * lax.rsqrt(var + 1e-5) * g1_ref[...] + b1_ref[...]

    qT = lax.dot_general(wq_ref[...], q0T, (((1,), (0,)), ((), ())),
                         preferred_element_type=jnp.float32)
    qT = qT * (1.0 / math.sqrt(C))
    logitsT = lax.dot_general(k_scr[...], qT, (((1,), (0,)), ((), ())),
                              preferred_element_type=jnp.float32)
    logitsT = logitsT + (mpT_ref[0] - 1.0) * 1e9

    iotaK = lax.broadcasted_iota(jnp.int32, (K, TN), 0).astype(jnp.float32)
    fK = jnp.float32(K)
    m1 = jnp.max(logitsT, axis=0, keepdims=True)
    i1f = jnp.min(jnp.where(logitsT == m1, iotaK, fK), axis=0, keepdims=True)
    excl = jnp.where(iotaK == i1f, -jnp.inf, logitsT)
    m2 = jnp.max(excl, axis=0, keepdims=True)
    i2f = jnp.min(jnp.where(excl == m2, iotaK, fK), axis=0, keepdims=True)
    i1 = i1f.astype(jnp.int32)
    i2 = i2f.astype(jnp.int32)

    e = jnp.exp(m2 - m1)
    w1v = m_row / (1.0 + e)
    w2v = e * w1v
    wT_out[0, pl.ds(0, 1), :] = w1v
    wT_out[0, pl.ds(1, 1), :] = w2v
    iT_out[0, pl.ds(0, 1), :] = i1
    iT_out[0, pl.ds(1, 1), :] = i2
    off = b * K
    f0_out[0] = i1 + off
    f1_out[0] = i2 + off
    w0_out[0] = w1v
    w1_out[0] = w2v


def _tc_attn(ridxf, nm, mpT, s_parent, wffc, Wq, Wk, Wv, g1c, b1c):
    B, N = ridxf.shape
    _, K, C = s_parent.shape
    TN = _TN
    grid = (B, N // TN)
    full2 = lambda shape: pl.BlockSpec(shape, lambda b, nb: (0, 0))
    plane = lambda dt: jax.ShapeDtypeStruct((B, 1, N), dt)
    plane_spec = lambda: pl.BlockSpec((1, 1, TN), lambda b, nb: (b, 0, nb))
    out_shape = [
        jax.ShapeDtypeStruct((B, K, C), jnp.float32),
        jax.ShapeDtypeStruct((B, 2, N), jnp.float32),
        jax.ShapeDtypeStruct((B, 2, N), jnp.int32),
        plane(jnp.int32), plane(jnp.int32),
        plane(jnp.float32), plane(jnp.float32),
    ]
    in_specs = [
        full2((B, N)),
        full2((B, N)),
        pl.BlockSpec((1, K, 1), lambda b, nb: (b, 0, 0)),
        pl.BlockSpec((1, K, C), lambda b, nb: (b, 0, 0)),
        full2((C // 2, 1)),
        full2((C, C)), full2((C, C)), full2((C, C)),
        full2((C, 1)), full2((C, 1)),
    ]
    out_specs = [
        pl.BlockSpec((1, K, C), lambda b, nb: (b, 0, 0)),
        pl.BlockSpec((1, 2, TN), lambda b, nb: (b, 0, nb)),
        pl.BlockSpec((1, 2, TN), lambda b, nb: (b, 0, nb)),
        plane_spec(), plane_spec(), plane_spec(), plane_spec(),
    ]
    return pl.pallas_call(
        _attn_body,
        grid=grid,
        in_specs=in_specs,
        out_specs=out_specs,
        out_shape=out_shape,
        scratch_shapes=[pltpu.VMEM((K, C), jnp.float32)],
        compiler_params=pltpu.CompilerParams(
            dimension_semantics=("arbitrary", "arbitrary")),
    )(ridxf, nm, mpT, s_parent, wffc, Wq, Wk, Wv, g1c, b1c)


def _wrep_body(w0_ref, w1_ref, o_ref):
    half = lax.broadcasted_iota(jnp.int32, (o_ref.shape[0], 32), 1) < 16
    for j in range(4):
        o_ref[:, pl.ds(32 * j, 32)] = jnp.where(
            half, w0_ref[:, j:j + 1], w1_ref[:, j:j + 1])


def _wrep(w0c, w1c):
    ROWS = w0c.shape[0]
    TRW = 2048
    return pl.pallas_call(
        _wrep_body,
        grid=(ROWS // (4 * TRW),),
        in_specs=[pl.BlockSpec((TRW, 4), lambda r: (r, 0)),
                  pl.BlockSpec((TRW, 4), lambda r: (r, 0))],
        out_specs=pl.BlockSpec((TRW, 128), lambda r: (r, 0)),
        out_shape=jax.ShapeDtypeStruct((ROWS // 4, 128), jnp.float32),
        compiler_params=pltpu.CompilerParams(
            dimension_semantics=("arbitrary",)),
    )(w0c.reshape(ROWS // 4, 4), w1c.reshape(ROWS // 4, 4))


def _sc_gather_combine(v2, i0, i1, wrep):
    ROWS = i0.shape[0]
    C = v2.shape[1]
    NW = 32
    RPW = ROWS // NW
    CH = _CH
    NCH = RPW // CH
    L = 16
    NCV = C // L

    mesh = plsc.VectorSubcoreMesh(core_axis_name="c", subcore_axis_name="s")

    NB = 3

    @functools.partial(
        pl.kernel, mesh=mesh,
        out_type=jax.ShapeDtypeStruct((ROWS, C), jnp.float32),
        scratch_types=[
            pltpu.VMEM((NB, CH), jnp.int32), pltpu.VMEM((NB, CH), jnp.int32),
            pltpu.VMEM((NB, CH, 32), jnp.float32),
            pltpu.VMEM((CH, C), jnp.float32), pltpu.VMEM((CH, C), jnp.float32),
            pltpu.VMEM((CH, C), jnp.float32), pltpu.VMEM((CH, C), jnp.float32),
            pltpu.VMEM((CH, C), jnp.float32), pltpu.VMEM((CH, C), jnp.float32),
            pltpu.SemaphoreType.DMA, pltpu.SemaphoreType.DMA,
            pltpu.SemaphoreType.DMA, pltpu.SemaphoreType.DMA,
            pltpu.SemaphoreType.DMA, pltpu.SemaphoreType.DMA,
            pltpu.SemaphoreType.DMA, pltpu.SemaphoreType.DMA,
            pltpu.SemaphoreType.DMA,
        ],
    )
    def body(v2_hbm, i0_hbm, i1_hbm, wr_hbm, o_hbm,
             ia_v, ib_v, wv_v, ra0, ra1, ra2, rb0, rb1, rb2,
             ga0, ga1, ga2, gb0, gb1, gb2, ws0, ws1, ws2):
        wid = lax.axis_index("s") * 2 + lax.axis_index("c")
        off = wid * RPW
        ra = (ra0, ra1, ra2)
        rb = (rb0, rb1, rb2)
        ga = (ga0, ga1, ga2)
        gb = (gb0, gb1, gb2)
        ws = (ws0, ws1, ws2)

        def start_gather(t):
            s = t % NB
            base = off + t * CH
            pltpu.sync_copy(i0_hbm.at[pl.ds(base, CH)], ia_v.at[s])
            pltpu.sync_copy(i1_hbm.at[pl.ds(base, CH)], ib_v.at[s])
            pltpu.sync_copy(wr_hbm.at[pl.ds(base, CH)], wv_v.at[s])
            pltpu.async_copy(v2_hbm.at[ia_v.at[s]], ra[s], ga[s])
            pltpu.async_copy(v2_hbm.at[ib_v.at[s]], rb[s], gb[s])

        def combine_writeback(t):
            s = t % NB
            base = off + t * CH
            pltpu.make_async_copy(v2_hbm.at[ia_v.at[s]], ra[s], ga[s]).wait()
            pltpu.make_async_copy(v2_hbm.at[ib_v.at[s]], rb[s], gb[s]).wait()

            def row(i, _):
                w0s = wv_v[s, i, pl.ds(0, L)]
                w1s = wv_v[s, i, pl.ds(L, L)]
                for cc in range(NCV):
                    sl = pl.ds(cc * L, L)
                    ra[s][i, sl] = w0s * ra[s][i, sl] + w1s * rb[s][i, sl]
                return 0

            lax.fori_loop(0, CH, row, 0)
            pltpu.async_copy(ra[s], o_hbm.at[pl.ds(base, CH)], ws[s])

        def wait_writeback(t):
            s = t % NB
            base = off + t * CH
            pltpu.make_async_copy(ra[s], o_hbm.at[pl.ds(base, CH)],
                                  ws[s]).wait()

        for t in range(NCH):
            if t >= NB:
                wait_writeback(t - NB)
            start_gather(t)
            if t >= 1:
                combine_writeback(t - 1)
        combine_writeback(NCH - 1)
        for t in range(max(NCH - NB, 0), NCH):
            wait_writeback(t)

    return body(v2, i0, i1, wrep)


def _ln_body(s_ref, g2_ref, b2_ref, o_ref):
    s0 = s_ref[...]
    mu = jnp.mean(s0, axis=-1, keepdims=True)
    var = jnp.mean((s0 - mu) ** 2, axis=-1, keepdims=True)
    o_ref[...] = (s0 - mu) * lax.rsqrt(var + 1e-5) * g2_ref[...] + b2_ref[...]


def _tc_ln(s0, g2r, b2r):
    ROWS, C = s0.shape
    TR = _TR
    return pl.pallas_call(
        _ln_body,
        grid=(ROWS // TR,),
        in_specs=[
            pl.BlockSpec((TR, C), lambda r: (r, 0)),
            pl.BlockSpec((1, C), lambda r: (0, 0)),
            pl.BlockSpec((1, C), lambda r: (0, 0)),
        ],
        out_specs=pl.BlockSpec((TR, C), lambda r: (r, 0)),
        out_shape=jax.ShapeDtypeStruct((ROWS, C), jnp.float32),
        compiler_params=pltpu.CompilerParams(
            dimension_semantics=("arbitrary",)),
    )(s0, g2r, b2r)


def kernel(s_parent, mask_parent, node_mask, res_idx, Wff, Wq, Wk, Wv,
           g1, b1, g2, b2):
    B, K, C = s_parent.shape
    N = res_idx.shape[1]
    ridxf = res_idx.astype(jnp.float32)
    nm = node_mask.astype(jnp.float32)
    mpT = mask_parent.astype(jnp.float32)[:, :, None]
    wffc = Wff.astype(jnp.float32).T
    g1c = g1[:, None]
    b1c = b1[:, None]

    v, wT, iT, f0, f1, w0, w1 = _tc_attn(ridxf, nm, mpT, s_parent, wffc,
                                         Wq, Wk, Wv, g1c, b1c)

    wrep = _wrep(w0.reshape(-1, 1), w1.reshape(-1, 1))
    s0 = _sc_gather_combine(v.reshape(B * K, C), f0.reshape(-1),
                            f1.reshape(-1), wrep.reshape(-1, 32))
    s_flat = _tc_ln(s0, g2[None, :], b2[None, :])

    s_fine = s_flat.reshape(B, N, C)
    B_local = jnp.transpose(wT, (0, 2, 1))
    parent_idx = jnp.transpose(iT, (0, 2, 1))
    return (s_fine, B_local, parent_idx, jnp.float32(0.0))


# speedup vs baseline: 1.0324x; 1.0324x over previous
import functools
import math

import jax
import jax.numpy as jnp
from jax import lax
from jax.experimental import pallas as pl
from jax.experimental.pallas import tpu as pltpu
from jax.experimental.pallas import tpu_sc as plsc

_TN = 2048
_TR = 1024
_CH = 64


def _attn_body(ridx_ref, nm_ref, mpT_ref, sp_ref, wff_ref, wq_ref, wk_ref,
               wv_ref, g1_ref, b1_ref,
               v_out, wT_out, iT_out, f0_out, f1_out, w0_out, w1_out, k_scr):
    b = pl.program_id(0)
    nb = pl.program_id(1)
    K, C = k_scr.shape
    TN = w0_out.shape[2]

    @pl.when(nb == 0)
    def _():
        sp = sp_ref[0]
        k_scr[...] = lax.dot_general(sp, wk_ref[...], (((1,), (1,)), ((), ())),
                                     preferred_element_type=jnp.float32)
        v_out[0] = lax.dot_general(sp, wv_ref[...], (((1,), (1,)), ((), ())),
                                   preferred_element_type=jnp.float32)

    Lb = jnp.maximum(jnp.sum(nm_ref[pl.ds(b, 1), :]), 1.0)
    denom = jnp.maximum(Lb - 1.0, 1.0)
    m_row = nm_ref[pl.ds(b, 1), pl.ds(nb * TN, TN)]
    pos = jnp.clip(ridx_ref[pl.ds(b, 1), pl.ds(nb * TN, TN)] / denom, 0.0, 1.0)
    projT = (2.0 * math.pi) * (wff_ref[...] * pos)
    q0T = jnp.concatenate([jnp.cos(projT), jnp.sin(projT)], axis=0)
    q0T = q0T * m_row
    mu = jnp.mean(q0T, axis=0, keepdims=True)
    var = jnp.mean((q0T - mu) ** 2, axis=0, keepdims=True)
    q0T = (q0T - mu) * lax.rsqrt(var + 1e-5) * g1_ref[...] + b1_ref[...]

    qT = lax.dot_general(wq_ref[...], q0T, (((1,), (0,)), ((), ())),
                         preferred_element_type=jnp.float32)
    qT = qT * (1.0 / math.sqrt(C))
    logitsT = lax.dot_general(k_scr[...], qT, (((1,), (0,)), ((), ())),
                              preferred_element_type=jnp.float32)
    logitsT = logitsT + (mpT_ref[0] - 1.0) * 1e9

    iotaK = lax.broadcasted_iota(jnp.int32, (K, TN), 0).astype(jnp.float32)
    fK = jnp.float32(K)
    m1 = jnp.max(logitsT, axis=0, keepdims=True)
    i1f = jnp.min(jnp.where(logitsT == m1, iotaK, fK), axis=0, keepdims=True)
    excl = jnp.where(iotaK == i1f, -jnp.inf, logitsT)
    m2 = jnp.max(excl, axis=0, keepdims=True)
    i2f = jnp.min(jnp.where(excl == m2, iotaK, fK), axis=0, keepdims=True)
    i1 = i1f.astype(jnp.int32)
    i2 = i2f.astype(jnp.int32)

    e = jnp.exp(m2 - m1)
    w1v = m_row / (1.0 + e)
    w2v = e * w1v
    wT_out[0, pl.ds(0, 1), :] = w1v
    wT_out[0, pl.ds(1, 1), :] = w2v
    iT_out[0, pl.ds(0, 1), :] = i1
    iT_out[0, pl.ds(1, 1), :] = i2
    off = b * K
    f0_out[0] = i1 + off
    f1_out[0] = i2 + off
    w0_out[0] = w1v
    w1_out[0] = w2v


def _tc_attn(ridxf, nm, mpT, s_parent, wffc, Wq, Wk, Wv, g1c, b1c):
    B, N = ridxf.shape
    _, K, C = s_parent.shape
    TN = _TN
    grid = (B, N // TN)
    full2 = lambda shape: pl.BlockSpec(shape, lambda b, nb: (0, 0))
    plane = lambda dt: jax.ShapeDtypeStruct((B, 1, N), dt)
    plane_spec = lambda: pl.BlockSpec((1, 1, TN), lambda b, nb: (b, 0, nb))
    out_shape = [
        jax.ShapeDtypeStruct((B, K, C), jnp.float32),
        jax.ShapeDtypeStruct((B, 2, N), jnp.float32),
        jax.ShapeDtypeStruct((B, 2, N), jnp.int32),
        plane(jnp.int32), plane(jnp.int32),
        plane(jnp.float32), plane(jnp.float32),
    ]
    in_specs = [
        full2((B, N)),
        full2((B, N)),
        pl.BlockSpec((1, K, 1), lambda b, nb: (b, 0, 0)),
        pl.BlockSpec((1, K, C), lambda b, nb: (b, 0, 0)),
        full2((C // 2, 1)),
        full2((C, C)), full2((C, C)), full2((C, C)),
        full2((C, 1)), full2((C, 1)),
    ]
    out_specs = [
        pl.BlockSpec((1, K, C), lambda b, nb: (b, 0, 0)),
        pl.BlockSpec((1, 2, TN), lambda b, nb: (b, 0, nb)),
        pl.BlockSpec((1, 2, TN), lambda b, nb: (b, 0, nb)),
        plane_spec(), plane_spec(), plane_spec(), plane_spec(),
    ]
    return pl.pallas_call(
        _attn_body,
        grid=grid,
        in_specs=in_specs,
        out_specs=out_specs,
        out_shape=out_shape,
        scratch_shapes=[pltpu.VMEM((K, C), jnp.float32)],
        compiler_params=pltpu.CompilerParams(
            dimension_semantics=("arbitrary", "arbitrary")),
    )(ridxf, nm, mpT, s_parent, wffc, Wq, Wk, Wv, g1c, b1c)


def _wrep_body(w0_ref, w1_ref, o_ref):
    lane = lax.broadcasted_iota(jnp.int32, o_ref.shape, 1)
    o_ref[...] = jnp.where(lane < 16, w0_ref[...], w1_ref[...])


def _wrep(w0c, w1c):
    ROWS = w0c.shape[0]
    TRW = 2048
    return pl.pallas_call(
        _wrep_body,
        grid=(ROWS // TRW,),
        in_specs=[pl.BlockSpec((TRW, 1), lambda r: (r, 0)),
                  pl.BlockSpec((TRW, 1), lambda r: (r, 0))],
        out_specs=pl.BlockSpec((TRW, 32), lambda r: (r, 0)),
        out_shape=jax.ShapeDtypeStruct((ROWS, 32), jnp.float32),
        compiler_params=pltpu.CompilerParams(
            dimension_semantics=("arbitrary",)),
    )(w0c, w1c)


def _sc_gather_combine(v2, i0, i1, wrep):
    ROWS = i0.shape[0]
    C = v2.shape[1]
    NW = 32
    RPW = ROWS // NW
    CH = _CH
    NCH = RPW // CH
    L = 16
    NCV = C // L

    mesh = plsc.VectorSubcoreMesh(core_axis_name="c", subcore_axis_name="s")

    NB = 3

    @functools.partial(
        pl.kernel, mesh=mesh,
        out_type=jax.ShapeDtypeStruct((ROWS, C), jnp.float32),
        scratch_types=[
            pltpu.VMEM((NB, CH), jnp.int32), pltpu.VMEM((NB, CH), jnp.int32),
            pltpu.VMEM((NB, CH, 32), jnp.float32),
            pltpu.VMEM((CH, C), jnp.float32), pltpu.VMEM((CH, C), jnp.float32),
            pltpu.VMEM((CH, C), jnp.float32), pltpu.VMEM((CH, C), jnp.float32),
            pltpu.VMEM((CH, C), jnp.float32), pltpu.VMEM((CH, C), jnp.float32),
            pltpu.SemaphoreType.DMA, pltpu.SemaphoreType.DMA,
            pltpu.SemaphoreType.DMA, pltpu.SemaphoreType.DMA,
            pltpu.SemaphoreType.DMA, pltpu.SemaphoreType.DMA,
            pltpu.SemaphoreType.DMA, pltpu.SemaphoreType.DMA,
            pltpu.SemaphoreType.DMA,
        ],
    )
    def body(v2_hbm, i0_hbm, i1_hbm, wr_hbm, o_hbm,
             ia_v, ib_v, wv_v, ra0, ra1, ra2, rb0, rb1, rb2,
             ga0, ga1, ga2, gb0, gb1, gb2, ws0, ws1, ws2):
        wid = lax.axis_index("s") * 2 + lax.axis_index("c")
        off = wid * RPW
        ra = (ra0, ra1, ra2)
        rb = (rb0, rb1, rb2)
        ga = (ga0, ga1, ga2)
        gb = (gb0, gb1, gb2)
        ws = (ws0, ws1, ws2)

        def start_gather(t):
            s = t % NB
            base = off + t * CH
            pltpu.sync_copy(i0_hbm.at[pl.ds(base, CH)], ia_v.at[s])
            pltpu.sync_copy(i1_hbm.at[pl.ds(base, CH)], ib_v.at[s])
            pltpu.sync_copy(wr_hbm.at[pl.ds(base, CH)], wv_v.at[s])
            pltpu.async_copy(v2_hbm.at[ia_v.at[s]], ra[s], ga[s])
            pltpu.async_copy(v2_hbm.at[ib_v.at[s]], rb[s], gb[s])

        def combine_writeback(t):
            s = t % NB
            base = off + t * CH
            pltpu.make_async_copy(v2_hbm.at[ia_v.at[s]], ra[s], ga[s]).wait()
            pltpu.make_async_copy(v2_hbm.at[ib_v.at[s]], rb[s], gb[s]).wait()

            def row(i, _):
                w0s = wv_v[s, i, pl.ds(0, L)]
                w1s = wv_v[s, i, pl.ds(L, L)]
                for cc in range(NCV):
                    sl = pl.ds(cc * L, L)
                    ra[s][i, sl] = w0s * ra[s][i, sl] + w1s * rb[s][i, sl]
                return 0

            lax.fori_loop(0, CH, row, 0)
            pltpu.async_copy(ra[s], o_hbm.at[pl.ds(base, CH)], ws[s])

        def wait_writeback(t):
            s = t % NB
            base = off + t * CH
            pltpu.make_async_copy(ra[s], o_hbm.at[pl.ds(base, CH)],
                                  ws[s]).wait()

        for t in range(NCH):
            if t >= NB:
                wait_writeback(t - NB)
            start_gather(t)
            if t >= 1:
                combine_writeback(t - 1)
        combine_writeback(NCH - 1)
        for t in range(max(NCH - NB, 0), NCH):
            wait_writeback(t)

    return body(v2, i0, i1, wrep)


def _ln_body(s_ref, g2_ref, b2_ref, o_ref):
    s0 = s_ref[...]
    mu = jnp.mean(s0, axis=-1, keepdims=True)
    var = jnp.mean((s0 - mu) ** 2, axis=-1, keepdims=True)
    o_ref[...] = (s0 - mu) * lax.rsqrt(var + 1e-5) * g2_ref[...] + b2_ref[...]


def _tc_ln(s0, g2r, b2r):
    ROWS, C = s0.shape
    TR = _TR
    return pl.pallas_call(
        _ln_body,
        grid=(ROWS // TR,),
        in_specs=[
            pl.BlockSpec((TR, C), lambda r: (r, 0)),
            pl.BlockSpec((1, C), lambda r: (0, 0)),
            pl.BlockSpec((1, C), lambda r: (0, 0)),
        ],
        out_specs=pl.BlockSpec((TR, C), lambda r: (r, 0)),
        out_shape=jax.ShapeDtypeStruct((ROWS, C), jnp.float32),
        compiler_params=pltpu.CompilerParams(
            dimension_semantics=("arbitrary",)),
    )(s0, g2r, b2r)


def kernel(s_parent, mask_parent, node_mask, res_idx, Wff, Wq, Wk, Wv,
           g1, b1, g2, b2):
    B, K, C = s_parent.shape
    N = res_idx.shape[1]
    ridxf = res_idx.astype(jnp.float32)
    nm = node_mask.astype(jnp.float32)
    mpT = mask_parent.astype(jnp.float32)[:, :, None]
    wffc = Wff.astype(jnp.float32).T
    g1c = g1[:, None]
    b1c = b1[:, None]

    v, wT, iT, f0, f1, w0, w1 = _tc_attn(ridxf, nm, mpT, s_parent, wffc,
                                         Wq, Wk, Wv, g1c, b1c)

    wrep = _wrep(w0.reshape(-1, 1), w1.reshape(-1, 1))
    s0 = _sc_gather_combine(v.reshape(B * K, C), f0.reshape(-1),
                            f1.reshape(-1), wrep)
    s_flat = _tc_ln(s0, g2[None, :], b2[None, :])

    s_fine = s_flat.reshape(B, N, C)
    B_local = jnp.transpose(wT, (0, 2, 1))
    parent_idx = jnp.transpose(iT, (0, 2, 1))
    return (s_fine, B_local, parent_idx, jnp.float32(0.0))
